# baseline (speedup 1.0000x reference)
import functools

import jax
import jax.numpy as jnp
from jax import lax
from jax.experimental import pallas as pl
from jax.experimental.pallas import tpu as pltpu
from jax.experimental.pallas import tpu_sc as plsc

NC = 2
NS = 16
NW = NC * NS
LANES = 16
BLK = 128

_N = 100000
_D = 16
_L = 3
_E = 3200000
_NBLK = _E // BLK
_RCHUNK = 448
_NPAD = -(-_N // (NW * _RCHUNK)) * (NW * _RCHUNK)
_RPS = _NPAD // NW
_NCH = _RPS // _RCHUNK
_SPS = _NPAD // NS
_ZROWS = _SPS // 16
_NZC = 16


def _mesh():
    return plsc.VectorSubcoreMesh(
        core_axis_name="c", subcore_axis_name="s", num_cores=NC, num_subcores=NS
    )


def _wid():
    c = lax.axis_index("c")
    s = lax.axis_index("s")
    return c, s, c * NS + s


def _vrsqrt(x):
    i = lax.bitcast_convert_type(x, jnp.int32)
    i = 0x5F3759DF - lax.shift_right_arithmetic(i, 1)
    y = lax.bitcast_convert_type(i, jnp.float32)
    for _ in range(3):
        y = y * (1.5 - 0.5 * x * y * y)
    return y


def _lane_sum_scalar(v):
    c = plsc.cumsum(v)
    return c[15]


def _softmax_weight(lw, l):
    a = [lw[i] for i in range(_L + 1)]
    m = a[0]
    for i in range(1, _L + 1):
        m = jnp.maximum(m, a[i])
    lane = lax.iota(jnp.int32, 16)
    e = jnp.where(lane < (_L + 1), jnp.exp(lw - m), 0.0)
    s = e[0]
    for i in range(1, _L + 1):
        s = s + e[i]
    return jnp.broadcast_to(e[l], (16,)) / jnp.broadcast_to(s, (16,))


def _edge_range(wid):
    per = _NBLK // NW
    rem = _NBLK % NW
    b0 = wid * per + jnp.minimum(wid, rem)
    nb = per + jnp.where(wid < rem, 1, 0)
    return b0, nb


_G = 4
_GE = _G * BLK
_NSB = _NBLK // _G


def _sb_range(wid):
    per = _NSB // NW
    rem = _NSB % NW
    g0 = wid * per + jnp.minimum(wid, rem)
    ng = per + jnp.where(wid < rem, 1, 0)
    return g0, ng


def _zero_fill(zb, nrows):
    z16 = jnp.zeros((16,), jnp.float32)

    def body(i, _):
        zb[i, :] = z16
        return 0

    lax.fori_loop(0, nrows, body, 0)


_DG = 8
_DNSB = _NBLK // _DG


def _deg_sb_range(wid):
    per = _DNSB // NW
    rem = _DNSB % NW
    g0 = wid * per + jnp.minimum(wid, rem)
    ng = per + jnp.where(wid < rem, 1, 0)
    return g0, ng


def _deg_body(dst_hbm, pdeg0_hbm, pdeg1_hbm, deg_sh, didx, ones_v, zb, isem, ssem):
    c, s, wid = _wid()
    z16 = jnp.zeros((16,), jnp.float32)
    one16 = jnp.full((16,), 1.0, jnp.float32)

    def zfill(i, _):
        zb[pl.ds(i * 16, 16)] = z16
        return 0

    lax.fori_loop(0, _ZROWS // 16, zfill, 0)
    for i in range(BLK // 16):
        ones_v[pl.ds(i * 16, 16)] = one16
    base = s * _SPS
    for k in range(_NZC):
        pltpu.sync_copy(zb, deg_sh.at[pl.ds(base + k * _ZROWS, _ZROWS)])
    plsc.subcore_barrier()

    g0, ng = _deg_sb_range(wid)
    pltpu.sync_copy(dst_hbm.at[pl.ds(g0 * _DG, _DG)], didx.at[0])

    def body(g, _):
        p = lax.rem(g, 3)

        @pl.when(g >= 1)
        def _():
            pltpu.make_async_copy(dst_hbm.at[pl.ds(0, _DG)], didx.at[p], isem).wait()

        @pl.when(g >= 2)
        def _():
            for _j in range(_DG):
                pltpu.make_async_copy(dst_hbm.at[0], didx.at[0, 0], ssem).wait()

        @pl.when(g + 1 < ng)
        def _():
            pltpu.async_copy(dst_hbm.at[pl.ds((g0 + g + 1) * _DG, _DG)],
                             didx.at[lax.rem(g + 1, 3)], isem)

        for j in range(_DG):
            pltpu.async_copy(ones_v, deg_sh.at[didx.at[p, j]], ssem, add=True)
        return 0

    lax.fori_loop(0, ng, body, 0)
    for _j in range(2 * _DG):
        pltpu.make_async_copy(dst_hbm.at[0], didx.at[0, 0], ssem).wait()
    plsc.subcore_barrier()

    for k in range(_NZC):
        sl = pl.ds(base + k * _ZROWS, _ZROWS)
        pltpu.sync_copy(deg_sh.at[sl], zb)

        @pl.when(c == 0)
        def _():
            pltpu.sync_copy(zb, pdeg0_hbm.at[sl])

        @pl.when(c == 1)
        def _():
            pltpu.sync_copy(zb, pdeg1_hbm.at[sl])


def _deg_call(dst):
    return pl.kernel(
        _deg_body,
        out_type=(
            jax.ShapeDtypeStruct((_NPAD,), jnp.float32),
            jax.ShapeDtypeStruct((_NPAD,), jnp.float32),
        ),
        mesh=_mesh(),
        compiler_params=pltpu.CompilerParams(use_tc_tiling_on_sc=False, needs_layout_passes=False),
        scratch_types=[
            pltpu.VMEM_SHARED((_NPAD,), jnp.float32),
            pltpu.VMEM((3, _DG, BLK), jnp.int32),
            pltpu.VMEM((BLK,), jnp.float32),
            pltpu.VMEM((_ZROWS,), jnp.float32),
            pltpu.SemaphoreType.DMA,
            pltpu.SemaphoreType.DMA,
        ],
    )(dst)


def _prep_body(pdeg0_hbm, pdeg1_hbm, x0_hbm, lw_hbm, dinv_hbm, y_hbm, agg_hbm,
               d0, d1, xc, yc, ac, lwv):
    c, s, wid = _wid()
    base = wid * _RPS
    pltpu.sync_copy(pdeg0_hbm.at[pl.ds(base, _RPS)], d0)
    pltpu.sync_copy(pdeg1_hbm.at[pl.ds(base, _RPS)], d1)
    pltpu.sync_copy(lw_hbm, lwv)
    w0 = _softmax_weight(lwv[...], 0)

    def dbody(i, _):
        sl = pl.ds(i * 16, 16)
        dsum = d0[sl] + d1[sl]
        inv = _vrsqrt(jnp.maximum(dsum, 1.0))
        d0[sl] = jnp.where(dsum >= 0.5, inv, 0.0)
        return 0

    lax.fori_loop(0, _RPS // 16, dbody, 0)
    pltpu.sync_copy(d0, dinv_hbm.at[pl.ds(base, _RPS)])
    for k in range(_NCH):
        rb = base + k * _RCHUNK
        pltpu.sync_copy(x0_hbm.at[pl.ds(rb, _RCHUNK)], xc)

        def gbody(g, _):
            r0 = g * 16
            dvec = d0[pl.ds(k * _RCHUNK + r0, 16)]
            for j in range(16):
                row = xc[r0 + j, :]
                d = dvec[j]
                yc[r0 + j, :] = row * d
                ac[r0 + j, :] = row * w0
            return 0

        lax.fori_loop(0, _RCHUNK // 16, gbody, 0)
        pltpu.sync_copy(yc, y_hbm.at[pl.ds(rb, _RCHUNK)])
        pltpu.sync_copy(ac, agg_hbm.at[pl.ds(rb, _RCHUNK)])


def _prep_call(pdeg0, pdeg1, x0, lw):
    return pl.kernel(
        _prep_body,
        out_type=(
            jax.ShapeDtypeStruct((_NPAD,), jnp.float32),
            jax.ShapeDtypeStruct((_NPAD, _D), jnp.float32),
            jax.ShapeDtypeStruct((_NPAD, _D), jnp.float32),
        ),
        mesh=_mesh(),
        compiler_params=pltpu.CompilerParams(use_tc_tiling_on_sc=False, needs_layout_passes=False),
        scratch_types=[
            pltpu.VMEM((_RPS,), jnp.float32),
            pltpu.VMEM((_RPS,), jnp.float32),
            pltpu.VMEM((_RCHUNK, _D), jnp.float32),
            pltpu.VMEM((_RCHUNK, _D), jnp.float32),
            pltpu.VMEM((_RCHUNK, _D), jnp.float32),
            pltpu.VMEM((LANES,), jnp.float32),
        ],
    )(pdeg0, pdeg1, x0, lw)


def _edge_body(src_hbm, dst_hbm, y_hbm, pacc0_hbm, pacc1_hbm,
               acc_sh, sidx, didx, rows, isem, gsem, ssem):
    c, s, wid = _wid()
    z16 = jnp.zeros((16,), jnp.float32)

    def zf(i, _):
        rows[0, i, :] = z16
        return 0

    lax.fori_loop(0, _ZROWS, zf, 0)
    base = s * _SPS
    for k in range(_NZC):
        pltpu.sync_copy(rows.at[0, pl.ds(0, _ZROWS)],
                        acc_sh.at[pl.ds(base + k * _ZROWS, _ZROWS)])
    plsc.subcore_barrier()
    g0, ng = _sb_range(wid)

    pltpu.sync_copy(src_hbm.at[pl.ds(g0 * _G, _G)], sidx.at[0])
    pltpu.sync_copy(dst_hbm.at[pl.ds(g0 * _G, _G)], didx.at[0])

    def _drain(sem, n):
        for _ in range(n):
            pltpu.make_async_copy(y_hbm.at[pl.ds(0, _GE)], rows.at[0], sem).wait()

    def body(g, _):
        p = lax.rem(g, 3)

        @pl.when(g >= 1)
        def _():
            pltpu.make_async_copy(src_hbm.at[pl.ds(0, _G)], sidx.at[p], isem).wait()
            pltpu.make_async_copy(dst_hbm.at[pl.ds(0, _G)], didx.at[p], isem).wait()

        gds = []
        for j in range(_G):
            gds.append(pltpu.async_copy(
                y_hbm.at[sidx.at[p, j]], rows.at[p, pl.ds(j * BLK, BLK)], gsem))

        @pl.when(g >= 2)
        def _():
            _drain(ssem, 1)

        @pl.when(g + 1 < ng)
        def _():
            q = lax.rem(g + 1, 3)
            pltpu.async_copy(src_hbm.at[pl.ds((g0 + g + 1) * _G, _G)],
                             sidx.at[q], isem)
            pltpu.async_copy(dst_hbm.at[pl.ds((g0 + g + 1) * _G, _G)],
                             didx.at[q], isem)

        for d in gds:
            d.wait()
        for j in range(_G):
            pltpu.async_copy(rows.at[p, pl.ds(j * BLK, BLK)],
                             acc_sh.at[didx.at[p, j]], ssem, add=True)
        return 0

    lax.fori_loop(0, ng, body, 0)
    _drain(ssem, 2)
    plsc.subcore_barrier()

    for k in range(_NZC):
        sl = pl.ds(base + k * _ZROWS, _ZROWS)
        pltpu.sync_copy(acc_sh.at[sl], rows.at[0, pl.ds(0, _ZROWS)])

        @pl.when(c == 0)
        def _():
            pltpu.sync_copy(rows.at[0, pl.ds(0, _ZROWS)], pacc0_hbm.at[sl])

        @pl.when(c == 1)
        def _():
            pltpu.sync_copy(rows.at[0, pl.ds(0, _ZROWS)], pacc1_hbm.at[sl])


def _edge_call(src, dst, y):
    return pl.kernel(
        _edge_body,
        out_type=(
            jax.ShapeDtypeStruct((_NPAD, _D), jnp.float32),
            jax.ShapeDtypeStruct((_NPAD, _D), jnp.float32),
        ),
        mesh=_mesh(),
        compiler_params=pltpu.CompilerParams(use_tc_tiling_on_sc=False, needs_layout_passes=False),
        scratch_types=[
            pltpu.VMEM_SHARED((_NPAD, _D), jnp.float32),
            pltpu.VMEM((3, _G, BLK), jnp.int32),
            pltpu.VMEM((3, _G, BLK), jnp.int32),
            pltpu.VMEM((3, _GE, _D), jnp.float32),
            pltpu.SemaphoreType.DMA,
            pltpu.SemaphoreType.DMA,
            pltpu.SemaphoreType.DMA,
        ],
    )(src, dst, y)


def _merge_body(l, pacc0_hbm, pacc1_hbm, dinv_hbm, agg_hbm, lw_hbm, y_hbm,
                aggo_hbm, p0, p1, ac, yc, dv, lwv):
    c, s, wid = _wid()
    pltpu.sync_copy(lw_hbm, lwv)
    wl = _softmax_weight(lwv[...], l)
    for k in range(_NCH):
        rb = wid * _RPS + k * _RCHUNK
        pltpu.sync_copy(pacc0_hbm.at[pl.ds(rb, _RCHUNK)], p0)
        pltpu.sync_copy(pacc1_hbm.at[pl.ds(rb, _RCHUNK)], p1)
        pltpu.sync_copy(dinv_hbm.at[pl.ds(rb, _RCHUNK)], dv)
        pltpu.sync_copy(agg_hbm.at[pl.ds(rb, _RCHUNK)], ac)

        def gbody(g, _):
            r0 = g * 16
            dvec = dv[pl.ds(r0, 16)]
            for j in range(16):
                d = dvec[j]
                x = (p0[r0 + j, :] + p1[r0 + j, :]) * d
                ac[r0 + j, :] = ac[r0 + j, :] + wl * x
                yc[r0 + j, :] = x * d
            return 0

        lax.fori_loop(0, _RCHUNK // 16, gbody, 0)
        pltpu.sync_copy(yc, y_hbm.at[pl.ds(rb, _RCHUNK)])
        pltpu.sync_copy(ac, aggo_hbm.at[pl.ds(rb, _RCHUNK)])


def _merge_call(l, pacc0, pacc1, dinv, agg, lw):
    return pl.kernel(
        functools.partial(_merge_body, l),
        out_type=(
            jax.ShapeDtypeStruct((_NPAD, _D), jnp.float32),
            jax.ShapeDtypeStruct((_NPAD, _D), jnp.float32),
        ),
        mesh=_mesh(),
        compiler_params=pltpu.CompilerParams(use_tc_tiling_on_sc=False, needs_layout_passes=False),
        scratch_types=[
            pltpu.VMEM((_RCHUNK, _D), jnp.float32),
            pltpu.VMEM((_RCHUNK, _D), jnp.float32),
            pltpu.VMEM((_RCHUNK, _D), jnp.float32),
            pltpu.VMEM((_RCHUNK, _D), jnp.float32),
            pltpu.VMEM((_RCHUNK,), jnp.float32),
            pltpu.VMEM((LANES,), jnp.float32),
        ],
    )(pacc0, pacc1, dinv, agg, lw)


def _final_body(pacc0_hbm, pacc1_hbm, dinv_hbm, agg_hbm, lw_hbm, out_hbm,
                p0, p1, ac, yc, dv, lwv):
    c, s, wid = _wid()
    pltpu.sync_copy(lw_hbm, lwv)
    wl = _softmax_weight(lwv[...], _L)
    for k in range(_NCH):
        rb = wid * _RPS + k * _RCHUNK
        pltpu.sync_copy(pacc0_hbm.at[pl.ds(rb, _RCHUNK)], p0)
        pltpu.sync_copy(pacc1_hbm.at[pl.ds(rb, _RCHUNK)], p1)
        pltpu.sync_copy(dinv_hbm.at[pl.ds(rb, _RCHUNK)], dv)
        pltpu.sync_copy(agg_hbm.at[pl.ds(rb, _RCHUNK)], ac)

        def gbody(g, _):
            r0 = g * 16
            dvec = dv[pl.ds(r0, 16)]
            for j in range(16):
                d = dvec[j]
                x = (p0[r0 + j, :] + p1[r0 + j, :]) * d
                a = ac[r0 + j, :] + wl * x
                ss = _lane_sum_scalar(a * a)
                ssv = jnp.broadcast_to(jnp.maximum(ss, 1e-24), (16,))
                yc[r0 + j, :] = a * _vrsqrt(ssv)
            return 0

        lax.fori_loop(0, _RCHUNK // 16, gbody, 0)
        pltpu.sync_copy(yc, out_hbm.at[pl.ds(rb, _RCHUNK)])


def _final_call(pacc0, pacc1, dinv, agg, lw):
    return pl.kernel(
        _final_body,
        out_type=jax.ShapeDtypeStruct((_NPAD, _D), jnp.float32),
        mesh=_mesh(),
        compiler_params=pltpu.CompilerParams(use_tc_tiling_on_sc=False, needs_layout_passes=False),
        scratch_types=[
            pltpu.VMEM((_RCHUNK, _D), jnp.float32),
            pltpu.VMEM((_RCHUNK, _D), jnp.float32),
            pltpu.VMEM((_RCHUNK, _D), jnp.float32),
            pltpu.VMEM((_RCHUNK, _D), jnp.float32),
            pltpu.VMEM((_RCHUNK,), jnp.float32),
            pltpu.VMEM((LANES,), jnp.float32),
        ],
    )(pacc0, pacc1, dinv, agg, lw)


def kernel(edge_index, embedding_weight, layer_weights):
    src2 = edge_index[0].reshape(_NBLK, BLK)
    dst2 = edge_index[1].reshape(_NBLK, BLK)
    x0 = jnp.zeros((_NPAD, _D), jnp.float32).at[:_N].set(embedding_weight)
    lw = jnp.pad(layer_weights.astype(jnp.float32),
                 (0, LANES - layer_weights.shape[0]))
    pdeg0, pdeg1 = _deg_call(dst2)
    dinv, y, agg = _prep_call(pdeg0, pdeg1, x0, lw)
    out = None
    for l in range(1, _L + 1):
        pacc0, pacc1 = _edge_call(src2, dst2, y)
        if l < _L:
            y, agg = _merge_call(l, pacc0, pacc1, dinv, agg, lw)
        else:
            out = _final_call(pacc0, pacc1, dinv, agg, lw)
    return out[:_N]


# speedup vs baseline: 1.0017x; 1.0017x over previous
import functools

import jax
import jax.numpy as jnp
from jax import lax
from jax.experimental import pallas as pl
from jax.experimental.pallas import tpu as pltpu
from jax.experimental.pallas import tpu_sc as plsc

NC = 2
NS = 16
NW = NC * NS
LANES = 16
BLK = 128

_N = 100000
_D = 16
_L = 3
_E = 3200000
_NBLK = _E // BLK
_RCHUNK = 448
_NPAD = -(-_N // (NW * _RCHUNK)) * (NW * _RCHUNK)
_RPS = _NPAD // NW
_NCH = _RPS // _RCHUNK
_SPS = _NPAD // NS
_ZROWS = _SPS // 16
_NZC = 16


def _mesh():
    return plsc.VectorSubcoreMesh(
        core_axis_name="c", subcore_axis_name="s", num_cores=NC, num_subcores=NS
    )


def _wid():
    c = lax.axis_index("c")
    s = lax.axis_index("s")
    return c, s, c * NS + s


def _vrsqrt(x):
    i = lax.bitcast_convert_type(x, jnp.int32)
    i = 0x5F3759DF - lax.shift_right_arithmetic(i, 1)
    y = lax.bitcast_convert_type(i, jnp.float32)
    for _ in range(3):
        y = y * (1.5 - 0.5 * x * y * y)
    return y


def _lane_sum_scalar(v):
    c = plsc.cumsum(v)
    return c[15]


def _softmax_weight(lw, l):
    a = [lw[i] for i in range(_L + 1)]
    m = a[0]
    for i in range(1, _L + 1):
        m = jnp.maximum(m, a[i])
    lane = lax.iota(jnp.int32, 16)
    e = jnp.where(lane < (_L + 1), jnp.exp(lw - m), 0.0)
    s = e[0]
    for i in range(1, _L + 1):
        s = s + e[i]
    return jnp.broadcast_to(e[l], (16,)) / jnp.broadcast_to(s, (16,))


def _edge_range(wid):
    per = _NBLK // NW
    rem = _NBLK % NW
    b0 = wid * per + jnp.minimum(wid, rem)
    nb = per + jnp.where(wid < rem, 1, 0)
    return b0, nb


_G = 4
_GE = _G * BLK
_NSB = _NBLK // _G


def _sb_range(wid):
    per = _NSB // NW
    rem = _NSB % NW
    g0 = wid * per + jnp.minimum(wid, rem)
    ng = per + jnp.where(wid < rem, 1, 0)
    return g0, ng


def _zero_fill(zb, nrows):
    z16 = jnp.zeros((16,), jnp.float32)

    def body(i, _):
        zb[i, :] = z16
        return 0

    lax.fori_loop(0, nrows, body, 0)


_DG = 8
_DNSB = _NBLK // _DG


def _deg_sb_range(wid):
    per = _DNSB // NW
    rem = _DNSB % NW
    g0 = wid * per + jnp.minimum(wid, rem)
    ng = per + jnp.where(wid < rem, 1, 0)
    return g0, ng


def _deg_body(dst_hbm, pdeg0_hbm, pdeg1_hbm, deg_sh, didx, ones_v, zb, isem, ssem):
    c, s, wid = _wid()
    z16 = jnp.zeros((16,), jnp.float32)
    one16 = jnp.full((16,), 1.0, jnp.float32)

    def zfill(i, _):
        zb[pl.ds(i * 16, 16)] = z16
        return 0

    lax.fori_loop(0, _ZROWS // 16, zfill, 0)
    for i in range(BLK // 16):
        ones_v[pl.ds(i * 16, 16)] = one16
    base = s * _SPS
    for k in range(_NZC):
        pltpu.sync_copy(zb, deg_sh.at[pl.ds(base + k * _ZROWS, _ZROWS)])
    plsc.subcore_barrier()

    g0, ng = _deg_sb_range(wid)
    pltpu.sync_copy(dst_hbm.at[pl.ds(g0 * _DG, _DG)], didx.at[0])

    def body(g, _):
        p = lax.rem(g, 3)

        @pl.when(g >= 1)
        def _():
            pltpu.make_async_copy(dst_hbm.at[pl.ds(0, _DG)], didx.at[p], isem).wait()

        @pl.when(g >= 2)
        def _():
            for _j in range(_DG):
                pltpu.make_async_copy(dst_hbm.at[0], didx.at[0, 0], ssem).wait()

        @pl.when(g + 1 < ng)
        def _():
            pltpu.async_copy(dst_hbm.at[pl.ds((g0 + g + 1) * _DG, _DG)],
                             didx.at[lax.rem(g + 1, 3)], isem)

        for j in range(_DG):
            pltpu.async_copy(ones_v, deg_sh.at[didx.at[p, j]], ssem, add=True)
        return 0

    lax.fori_loop(0, ng, body, 0)
    for _j in range(2 * _DG):
        pltpu.make_async_copy(dst_hbm.at[0], didx.at[0, 0], ssem).wait()
    plsc.subcore_barrier()

    for k in range(_NZC):
        sl = pl.ds(base + k * _ZROWS, _ZROWS)
        pltpu.sync_copy(deg_sh.at[sl], zb)

        @pl.when(c == 0)
        def _():
            pltpu.sync_copy(zb, pdeg0_hbm.at[sl])

        @pl.when(c == 1)
        def _():
            pltpu.sync_copy(zb, pdeg1_hbm.at[sl])


def _deg_call(dst):
    return pl.kernel(
        _deg_body,
        out_type=(
            jax.ShapeDtypeStruct((_NPAD,), jnp.float32),
            jax.ShapeDtypeStruct((_NPAD,), jnp.float32),
        ),
        mesh=_mesh(),
        compiler_params=pltpu.CompilerParams(use_tc_tiling_on_sc=False, needs_layout_passes=False),
        scratch_types=[
            pltpu.VMEM_SHARED((_NPAD,), jnp.float32),
            pltpu.VMEM((3, _DG, BLK), jnp.int32),
            pltpu.VMEM((BLK,), jnp.float32),
            pltpu.VMEM((_ZROWS,), jnp.float32),
            pltpu.SemaphoreType.DMA,
            pltpu.SemaphoreType.DMA,
        ],
    )(dst)


def _prep_body(pdeg0_hbm, pdeg1_hbm, x0_hbm, lw_hbm, dinv_hbm, y_hbm, agg_hbm,
               d0, d1, xc, yc, ac, lwv):
    c, s, wid = _wid()
    base = wid * _RPS
    pltpu.sync_copy(pdeg0_hbm.at[pl.ds(base, _RPS)], d0)
    pltpu.sync_copy(pdeg1_hbm.at[pl.ds(base, _RPS)], d1)
    pltpu.sync_copy(lw_hbm, lwv)
    w0 = _softmax_weight(lwv[...], 0)

    def dbody(i, _):
        sl = pl.ds(i * 16, 16)
        dsum = d0[sl] + d1[sl]
        inv = _vrsqrt(jnp.maximum(dsum, 1.0))
        d0[sl] = jnp.where(dsum >= 0.5, inv, 0.0)
        return 0

    lax.fori_loop(0, _RPS // 16, dbody, 0)
    pltpu.sync_copy(d0, dinv_hbm.at[pl.ds(base, _RPS)])
    for k in range(_NCH):
        rb = base + k * _RCHUNK
        pltpu.sync_copy(x0_hbm.at[pl.ds(rb, _RCHUNK)], xc)

        def gbody(g, _):
            r0 = g * 16
            dvec = d0[pl.ds(k * _RCHUNK + r0, 16)]
            for j in range(16):
                row = xc[r0 + j, :]
                d = dvec[j]
                yc[r0 + j, :] = row * d
                ac[r0 + j, :] = row * w0
            return 0

        lax.fori_loop(0, _RCHUNK // 16, gbody, 0)
        pltpu.sync_copy(yc, y_hbm.at[pl.ds(rb, _RCHUNK)])
        pltpu.sync_copy(ac, agg_hbm.at[pl.ds(rb, _RCHUNK)])


def _prep_call(pdeg0, pdeg1, x0, lw):
    return pl.kernel(
        _prep_body,
        out_type=(
            jax.ShapeDtypeStruct((_NPAD,), jnp.float32),
            jax.ShapeDtypeStruct((_NPAD, _D), jnp.float32),
            jax.ShapeDtypeStruct((_NPAD, _D), jnp.float32),
        ),
        mesh=_mesh(),
        compiler_params=pltpu.CompilerParams(use_tc_tiling_on_sc=False, needs_layout_passes=False),
        scratch_types=[
            pltpu.VMEM((_RPS,), jnp.float32),
            pltpu.VMEM((_RPS,), jnp.float32),
            pltpu.VMEM((_RCHUNK, _D), jnp.float32),
            pltpu.VMEM((_RCHUNK, _D), jnp.float32),
            pltpu.VMEM((_RCHUNK, _D), jnp.float32),
            pltpu.VMEM((LANES,), jnp.float32),
        ],
    )(pdeg0, pdeg1, x0, lw)


def _edge_body(src_hbm, dst_hbm, y_hbm, pacc0_hbm, pacc1_hbm,
               acc_sh, sidx, didx, rows, isem, gsem, ssem):
    c, s, wid = _wid()
    z16 = jnp.zeros((16,), jnp.float32)

    def zf(i, _):
        rows[0, i, :] = z16
        return 0

    lax.fori_loop(0, _ZROWS, zf, 0)
    base = s * _SPS
    for k in range(_NZC):
        pltpu.sync_copy(rows.at[0, pl.ds(0, _ZROWS)],
                        acc_sh.at[pl.ds(base + k * _ZROWS, _ZROWS)])
    plsc.subcore_barrier()
    g0, ng = _sb_range(wid)

    pltpu.sync_copy(src_hbm.at[pl.ds(g0 * _G, _G)], sidx.at[0])
    pltpu.sync_copy(dst_hbm.at[pl.ds(g0 * _G, _G)], didx.at[0])

    def _drain(sem, n):
        for _ in range(n):
            pltpu.make_async_copy(y_hbm.at[pl.ds(0, _GE)], rows.at[0], sem).wait()

    def body(g, _):
        p = lax.rem(g, 3)

        @pl.when(g >= 1)
        def _():
            pltpu.make_async_copy(src_hbm.at[pl.ds(0, _G)], sidx.at[p], isem).wait()
            pltpu.make_async_copy(dst_hbm.at[pl.ds(0, _G)], didx.at[p], isem).wait()

        gds = []
        for j in range(_G):
            gds.append(pltpu.async_copy(
                y_hbm.at[sidx.at[p, j]], rows.at[p, pl.ds(j * BLK, BLK)], gsem))

        @pl.when(g >= 2)
        def _():
            _drain(ssem, 1)

        @pl.when(g + 1 < ng)
        def _():
            q = lax.rem(g + 1, 3)
            pltpu.async_copy(src_hbm.at[pl.ds((g0 + g + 1) * _G, _G)],
                             sidx.at[q], isem)
            pltpu.async_copy(dst_hbm.at[pl.ds((g0 + g + 1) * _G, _G)],
                             didx.at[q], isem)

        for j in range(_G):
            gds[j].wait()
            pltpu.async_copy(rows.at[p, pl.ds(j * BLK, BLK)],
                             acc_sh.at[didx.at[p, j]], ssem, add=True)
        return 0

    lax.fori_loop(0, ng, body, 0)
    _drain(ssem, 2)
    plsc.subcore_barrier()

    for k in range(_NZC):
        sl = pl.ds(base + k * _ZROWS, _ZROWS)
        pltpu.sync_copy(acc_sh.at[sl], rows.at[0, pl.ds(0, _ZROWS)])

        @pl.when(c == 0)
        def _():
            pltpu.sync_copy(rows.at[0, pl.ds(0, _ZROWS)], pacc0_hbm.at[sl])

        @pl.when(c == 1)
        def _():
            pltpu.sync_copy(rows.at[0, pl.ds(0, _ZROWS)], pacc1_hbm.at[sl])


def _edge_call(src, dst, y):
    return pl.kernel(
        _edge_body,
        out_type=(
            jax.ShapeDtypeStruct((_NPAD, _D), jnp.float32),
            jax.ShapeDtypeStruct((_NPAD, _D), jnp.float32),
        ),
        mesh=_mesh(),
        compiler_params=pltpu.CompilerParams(use_tc_tiling_on_sc=False, needs_layout_passes=False),
        scratch_types=[
            pltpu.VMEM_SHARED((_NPAD, _D), jnp.float32),
            pltpu.VMEM((3, _G, BLK), jnp.int32),
            pltpu.VMEM((3, _G, BLK), jnp.int32),
            pltpu.VMEM((3, _GE, _D), jnp.float32),
            pltpu.SemaphoreType.DMA,
            pltpu.SemaphoreType.DMA,
            pltpu.SemaphoreType.DMA,
        ],
    )(src, dst, y)


def _merge_body(l, pacc0_hbm, pacc1_hbm, dinv_hbm, agg_hbm, lw_hbm, y_hbm,
                aggo_hbm, p0, p1, ac, yc, dv, lwv):
    c, s, wid = _wid()
    pltpu.sync_copy(lw_hbm, lwv)
    wl = _softmax_weight(lwv[...], l)
    for k in range(_NCH):
        rb = wid * _RPS + k * _RCHUNK
        pltpu.sync_copy(pacc0_hbm.at[pl.ds(rb, _RCHUNK)], p0)
        pltpu.sync_copy(pacc1_hbm.at[pl.ds(rb, _RCHUNK)], p1)
        pltpu.sync_copy(dinv_hbm.at[pl.ds(rb, _RCHUNK)], dv)
        pltpu.sync_copy(agg_hbm.at[pl.ds(rb, _RCHUNK)], ac)

        def gbody(g, _):
            r0 = g * 16
            dvec = dv[pl.ds(r0, 16)]
            for j in range(16):
                d = dvec[j]
                x = (p0[r0 + j, :] + p1[r0 + j, :]) * d
                ac[r0 + j, :] = ac[r0 + j, :] + wl * x
                yc[r0 + j, :] = x * d
            return 0

        lax.fori_loop(0, _RCHUNK // 16, gbody, 0)
        pltpu.sync_copy(yc, y_hbm.at[pl.ds(rb, _RCHUNK)])
        pltpu.sync_copy(ac, aggo_hbm.at[pl.ds(rb, _RCHUNK)])


def _merge_call(l, pacc0, pacc1, dinv, agg, lw):
    return pl.kernel(
        functools.partial(_merge_body, l),
        out_type=(
            jax.ShapeDtypeStruct((_NPAD, _D), jnp.float32),
            jax.ShapeDtypeStruct((_NPAD, _D), jnp.float32),
        ),
        mesh=_mesh(),
        compiler_params=pltpu.CompilerParams(use_tc_tiling_on_sc=False, needs_layout_passes=False),
        scratch_types=[
            pltpu.VMEM((_RCHUNK, _D), jnp.float32),
            pltpu.VMEM((_RCHUNK, _D), jnp.float32),
            pltpu.VMEM((_RCHUNK, _D), jnp.float32),
            pltpu.VMEM((_RCHUNK, _D), jnp.float32),
            pltpu.VMEM((_RCHUNK,), jnp.float32),
            pltpu.VMEM((LANES,), jnp.float32),
        ],
    )(pacc0, pacc1, dinv, agg, lw)


def _final_body(pacc0_hbm, pacc1_hbm, dinv_hbm, agg_hbm, lw_hbm, out_hbm,
                p0, p1, ac, yc, dv, lwv):
    c, s, wid = _wid()
    pltpu.sync_copy(lw_hbm, lwv)
    wl = _softmax_weight(lwv[...], _L)
    for k in range(_NCH):
        rb = wid * _RPS + k * _RCHUNK
        pltpu.sync_copy(pacc0_hbm.at[pl.ds(rb, _RCHUNK)], p0)
        pltpu.sync_copy(pacc1_hbm.at[pl.ds(rb, _RCHUNK)], p1)
        pltpu.sync_copy(dinv_hbm.at[pl.ds(rb, _RCHUNK)], dv)
        pltpu.sync_copy(agg_hbm.at[pl.ds(rb, _RCHUNK)], ac)

        def gbody(g, _):
            r0 = g * 16
            dvec = dv[pl.ds(r0, 16)]
            for j in range(16):
                d = dvec[j]
                x = (p0[r0 + j, :] + p1[r0 + j, :]) * d
                a = ac[r0 + j, :] + wl * x
                ss = _lane_sum_scalar(a * a)
                ssv = jnp.broadcast_to(jnp.maximum(ss, 1e-24), (16,))
                yc[r0 + j, :] = a * _vrsqrt(ssv)
            return 0

        lax.fori_loop(0, _RCHUNK // 16, gbody, 0)
        pltpu.sync_copy(yc, out_hbm.at[pl.ds(rb, _RCHUNK)])


def _final_call(pacc0, pacc1, dinv, agg, lw):
    return pl.kernel(
        _final_body,
        out_type=jax.ShapeDtypeStruct((_NPAD, _D), jnp.float32),
        mesh=_mesh(),
        compiler_params=pltpu.CompilerParams(use_tc_tiling_on_sc=False, needs_layout_passes=False),
        scratch_types=[
            pltpu.VMEM((_RCHUNK, _D), jnp.float32),
            pltpu.VMEM((_RCHUNK, _D), jnp.float32),
            pltpu.VMEM((_RCHUNK, _D), jnp.float32),
            pltpu.VMEM((_RCHUNK, _D), jnp.float32),
            pltpu.VMEM((_RCHUNK,), jnp.float32),
            pltpu.VMEM((LANES,), jnp.float32),
        ],
    )(pacc0, pacc1, dinv, agg, lw)


def kernel(edge_index, embedding_weight, layer_weights):
    src2 = edge_index[0].reshape(_NBLK, BLK)
    dst2 = edge_index[1].reshape(_NBLK, BLK)
    x0 = jnp.zeros((_NPAD, _D), jnp.float32).at[:_N].set(embedding_weight)
    lw = jnp.pad(layer_weights.astype(jnp.float32),
                 (0, LANES - layer_weights.shape[0]))
    pdeg0, pdeg1 = _deg_call(dst2)
    dinv, y, agg = _prep_call(pdeg0, pdeg1, x0, lw)
    out = None
    for l in range(1, _L + 1):
        pacc0, pacc1 = _edge_call(src2, dst2, y)
        if l < _L:
            y, agg = _merge_call(l, pacc0, pacc1, dinv, agg, lw)
        else:
            out = _final_call(pacc0, pacc1, dinv, agg, lw)
    return out[:_N]


# speedup vs baseline: 1.0555x; 1.0537x over previous
import functools

import jax
import jax.numpy as jnp
from jax import lax
from jax.experimental import pallas as pl
from jax.experimental.pallas import tpu as pltpu
from jax.experimental.pallas import tpu_sc as plsc

NC = 2
NS = 16
NW = NC * NS
LANES = 16
BLK = 128

_N = 100000
_D = 16
_L = 3
_E = 3200000
_NBLK = _E // BLK
_RCHUNK = 448
_NPAD = -(-_N // (NW * _RCHUNK)) * (NW * _RCHUNK)
_RPS = _NPAD // NW
_NCH = _RPS // _RCHUNK
_SPS = _NPAD // NS
_ZROWS = _SPS // 16
_NZC = 16


def _mesh():
    return plsc.VectorSubcoreMesh(
        core_axis_name="c", subcore_axis_name="s", num_cores=NC, num_subcores=NS
    )


def _wid():
    c = lax.axis_index("c")
    s = lax.axis_index("s")
    return c, s, c * NS + s


def _vrsqrt(x):
    i = lax.bitcast_convert_type(x, jnp.int32)
    i = 0x5F3759DF - lax.shift_right_arithmetic(i, 1)
    y = lax.bitcast_convert_type(i, jnp.float32)
    for _ in range(3):
        y = y * (1.5 - 0.5 * x * y * y)
    return y


def _lane_sum_scalar(v):
    c = plsc.cumsum(v)
    return c[15]


def _softmax_weight(lw, l):
    a = [lw[i] for i in range(_L + 1)]
    m = a[0]
    for i in range(1, _L + 1):
        m = jnp.maximum(m, a[i])
    lane = lax.iota(jnp.int32, 16)
    e = jnp.where(lane < (_L + 1), jnp.exp(lw - m), 0.0)
    s = e[0]
    for i in range(1, _L + 1):
        s = s + e[i]
    return jnp.broadcast_to(e[l], (16,)) / jnp.broadcast_to(s, (16,))


def _edge_range(wid):
    per = _NBLK // NW
    rem = _NBLK % NW
    b0 = wid * per + jnp.minimum(wid, rem)
    nb = per + jnp.where(wid < rem, 1, 0)
    return b0, nb


_G = 4
_GE = _G * BLK
_NSB = _NBLK // _G


def _sb_range(wid):
    per = _NSB // NW
    rem = _NSB % NW
    g0 = wid * per + jnp.minimum(wid, rem)
    ng = per + jnp.where(wid < rem, 1, 0)
    return g0, ng


def _zero_fill(zb, nrows):
    z16 = jnp.zeros((16,), jnp.float32)

    def body(i, _):
        zb[i, :] = z16
        return 0

    lax.fori_loop(0, nrows, body, 0)


_DG = 8
_DNSB = _NBLK // _DG


def _deg_sb_range(wid):
    per = _DNSB // NW
    rem = _DNSB % NW
    g0 = wid * per + jnp.minimum(wid, rem)
    ng = per + jnp.where(wid < rem, 1, 0)
    return g0, ng


def _deg_body(dst_hbm, pdeg0_hbm, pdeg1_hbm, deg_sh, didx, ones_v, zb, isem, ssem):
    c, s, wid = _wid()
    z16 = jnp.zeros((16,), jnp.float32)
    one16 = jnp.full((16,), 1.0, jnp.float32)

    def zfill(i, _):
        zb[pl.ds(i * 16, 16)] = z16
        return 0

    lax.fori_loop(0, _ZROWS // 16, zfill, 0)
    for i in range(BLK // 16):
        ones_v[pl.ds(i * 16, 16)] = one16
    base = s * _SPS
    for k in range(_NZC):
        pltpu.sync_copy(zb, deg_sh.at[pl.ds(base + k * _ZROWS, _ZROWS)])
    plsc.subcore_barrier()

    g0, ng = _deg_sb_range(wid)
    pltpu.sync_copy(dst_hbm.at[pl.ds(g0 * _DG, _DG)], didx.at[0])

    def body(g, _):
        p = lax.rem(g, 3)

        @pl.when(g >= 1)
        def _():
            pltpu.make_async_copy(dst_hbm.at[pl.ds(0, _DG)], didx.at[p], isem).wait()

        @pl.when(g >= 2)
        def _():
            for _j in range(_DG):
                pltpu.make_async_copy(dst_hbm.at[0], didx.at[0, 0], ssem).wait()

        @pl.when(g + 1 < ng)
        def _():
            pltpu.async_copy(dst_hbm.at[pl.ds((g0 + g + 1) * _DG, _DG)],
                             didx.at[lax.rem(g + 1, 3)], isem)

        for j in range(_DG):
            pltpu.async_copy(ones_v, deg_sh.at[didx.at[p, j]], ssem, add=True)
        return 0

    lax.fori_loop(0, ng, body, 0)
    for _j in range(2 * _DG):
        pltpu.make_async_copy(dst_hbm.at[0], didx.at[0, 0], ssem).wait()
    plsc.subcore_barrier()

    for k in range(_NZC):
        sl = pl.ds(base + k * _ZROWS, _ZROWS)
        pltpu.sync_copy(deg_sh.at[sl], zb)

        @pl.when(c == 0)
        def _():
            pltpu.sync_copy(zb, pdeg0_hbm.at[sl])

        @pl.when(c == 1)
        def _():
            pltpu.sync_copy(zb, pdeg1_hbm.at[sl])


def _deg_call(dst):
    return pl.kernel(
        _deg_body,
        out_type=(
            jax.ShapeDtypeStruct((_NPAD,), jnp.float32),
            jax.ShapeDtypeStruct((_NPAD,), jnp.float32),
        ),
        mesh=_mesh(),
        compiler_params=pltpu.CompilerParams(use_tc_tiling_on_sc=False, needs_layout_passes=False),
        scratch_types=[
            pltpu.VMEM_SHARED((_NPAD,), jnp.float32),
            pltpu.VMEM((3, _DG, BLK), jnp.int32),
            pltpu.VMEM((BLK,), jnp.float32),
            pltpu.VMEM((_ZROWS,), jnp.float32),
            pltpu.SemaphoreType.DMA,
            pltpu.SemaphoreType.DMA,
        ],
    )(dst)


def _prep_body(pdeg0_hbm, pdeg1_hbm, x0_hbm, lw_hbm, dinv_hbm, y_hbm, agg_hbm,
               d0, d1, xc, yc, ac, lwv):
    c, s, wid = _wid()
    base = wid * _RPS
    pltpu.sync_copy(pdeg0_hbm.at[pl.ds(base, _RPS)], d0)
    pltpu.sync_copy(pdeg1_hbm.at[pl.ds(base, _RPS)], d1)
    pltpu.sync_copy(lw_hbm, lwv)
    w0 = _softmax_weight(lwv[...], 0)

    def dbody(i, _):
        sl = pl.ds(i * 16, 16)
        dsum = d0[sl] + d1[sl]
        inv = _vrsqrt(jnp.maximum(dsum, 1.0))
        d0[sl] = jnp.where(dsum >= 0.5, inv, 0.0)
        return 0

    lax.fori_loop(0, _RPS // 16, dbody, 0)
    pltpu.sync_copy(d0, dinv_hbm.at[pl.ds(base, _RPS)])
    for k in range(_NCH):
        rb = base + k * _RCHUNK
        pltpu.sync_copy(x0_hbm.at[pl.ds(rb, _RCHUNK)], xc)

        def gbody(g, _):
            r0 = g * 16
            dvec = d0[pl.ds(k * _RCHUNK + r0, 16)]
            for j in range(16):
                row = xc[r0 + j, :]
                d = dvec[j]
                yc[r0 + j, :] = row * d
                ac[r0 + j, :] = row * w0
            return 0

        lax.fori_loop(0, _RCHUNK // 16, gbody, 0)
        pltpu.sync_copy(yc, y_hbm.at[pl.ds(rb, _RCHUNK)])
        pltpu.sync_copy(ac, agg_hbm.at[pl.ds(rb, _RCHUNK)])


def _prep_call(pdeg0, pdeg1, x0, lw):
    return pl.kernel(
        _prep_body,
        out_type=(
            jax.ShapeDtypeStruct((_NPAD,), jnp.float32),
            jax.ShapeDtypeStruct((_NPAD, _D), jnp.float32),
            jax.ShapeDtypeStruct((_NPAD, _D), jnp.float32),
        ),
        mesh=_mesh(),
        compiler_params=pltpu.CompilerParams(use_tc_tiling_on_sc=False, needs_layout_passes=False),
        scratch_types=[
            pltpu.VMEM((_RPS,), jnp.float32),
            pltpu.VMEM((_RPS,), jnp.float32),
            pltpu.VMEM((_RCHUNK, _D), jnp.float32),
            pltpu.VMEM((_RCHUNK, _D), jnp.float32),
            pltpu.VMEM((_RCHUNK, _D), jnp.float32),
            pltpu.VMEM((LANES,), jnp.float32),
        ],
    )(pdeg0, pdeg1, x0, lw)


def _edge_body(src_hbm, dst_hbm, y_hbm, pacc0_hbm, pacc1_hbm,
               acc_sh, sidx, didx, rows, isem, gsem, ssem):
    c, s, wid = _wid()
    z16 = jnp.zeros((16,), jnp.float32)

    def zf(i, _):
        rows[0, i, :] = z16
        return 0

    lax.fori_loop(0, _ZROWS, zf, 0)
    base = s * _SPS
    for k in range(_NZC):
        pltpu.sync_copy(rows.at[0, pl.ds(0, _ZROWS)],
                        acc_sh.at[pl.ds(base + k * _ZROWS, _ZROWS)])
    plsc.subcore_barrier()
    g0, ng = _sb_range(wid)

    pltpu.sync_copy(src_hbm.at[pl.ds(g0 * _G, _G)], sidx.at[0])
    pltpu.sync_copy(dst_hbm.at[pl.ds(g0 * _G, _G)], didx.at[0])

    def _drain(sem, n):
        for _ in range(n):
            pltpu.make_async_copy(y_hbm.at[pl.ds(0, _GE)], rows.at[0], sem).wait()

    def body(g, _):
        p = lax.rem(g, 3)

        @pl.when(g >= 1)
        def _():
            pltpu.make_async_copy(src_hbm.at[pl.ds(0, _G)], sidx.at[p], isem).wait()
            pltpu.make_async_copy(dst_hbm.at[pl.ds(0, _G)], didx.at[p], isem).wait()

        gds = []
        for j in range(_G):
            gds.append(pltpu.async_copy(
                y_hbm.at[sidx.at[p, j]], rows.at[p, pl.ds(j * BLK, BLK)], gsem))

        @pl.when(g >= 2)
        def _():
            _drain(ssem, 1)

        @pl.when(g + 1 < ng)
        def _():
            q = lax.rem(g + 1, 3)
            pltpu.async_copy(src_hbm.at[pl.ds((g0 + g + 1) * _G, _G)],
                             sidx.at[q], isem)
            pltpu.async_copy(dst_hbm.at[pl.ds((g0 + g + 1) * _G, _G)],
                             didx.at[q], isem)

        for j in range(_G):
            gds[j].wait()
            pltpu.async_copy(rows.at[p, pl.ds(j * BLK, BLK)],
                             acc_sh.at[didx.at[p, j]], ssem, add=True)
        return 0

    lax.fori_loop(0, ng, body, 0)
    _drain(ssem, 2)
    plsc.subcore_barrier()

    for k in range(_NZC):
        sl = pl.ds(base + k * _ZROWS, _ZROWS)
        pltpu.sync_copy(acc_sh.at[sl], rows.at[0, pl.ds(0, _ZROWS)])

        @pl.when(c == 0)
        def _():
            pltpu.sync_copy(rows.at[0, pl.ds(0, _ZROWS)], pacc0_hbm.at[sl])

        @pl.when(c == 1)
        def _():
            pltpu.sync_copy(rows.at[0, pl.ds(0, _ZROWS)], pacc1_hbm.at[sl])


def _edge_call(src, dst, y):
    return pl.kernel(
        _edge_body,
        out_type=(
            jax.ShapeDtypeStruct((_NPAD, _D), jnp.float32),
            jax.ShapeDtypeStruct((_NPAD, _D), jnp.float32),
        ),
        mesh=_mesh(),
        compiler_params=pltpu.CompilerParams(use_tc_tiling_on_sc=False, needs_layout_passes=False),
        scratch_types=[
            pltpu.VMEM_SHARED((_NPAD, _D), jnp.float32),
            pltpu.VMEM((3, _G, BLK), jnp.int32),
            pltpu.VMEM((3, _G, BLK), jnp.int32),
            pltpu.VMEM((3, _GE, _D), jnp.float32),
            pltpu.SemaphoreType.DMA,
            pltpu.SemaphoreType.DMA,
            pltpu.SemaphoreType.DMA,
        ],
    )(src, dst, y)


def _merge_body(l, pacc0_hbm, pacc1_hbm, dinv_hbm, agg_hbm, lw_hbm, y_hbm,
                aggo_hbm, p0, p1, ac, yc, dv, lwv, lsem, osem):
    c, s, wid = _wid()
    pltpu.sync_copy(lw_hbm, lwv)
    wl = _softmax_weight(lwv[...], l)

    def fire_loads(k, p):
        rb = wid * _RPS + k * _RCHUNK
        sl = pl.ds(rb, _RCHUNK)
        return [pltpu.async_copy(pacc0_hbm.at[sl], p0.at[p], lsem),
                pltpu.async_copy(pacc1_hbm.at[sl], p1.at[p], lsem),
                pltpu.async_copy(dinv_hbm.at[sl], dv.at[p], lsem),
                pltpu.async_copy(agg_hbm.at[sl], ac.at[p], lsem)]

    ldesc = [None, None]
    sdesc = [None, None]
    ldesc[0] = fire_loads(0, 0)
    for k in range(_NCH):
        p = k & 1
        for d_ in ldesc[p]:
            d_.wait()
        if k >= 1:
            for d_ in sdesc[1 - p]:
                d_.wait()
        if k + 1 < _NCH:
            ldesc[1 - p] = fire_loads(k + 1, 1 - p)

        def gbody(g, _):
            r0 = g * 16
            dvec = dv[p, pl.ds(r0, 16)]
            for j in range(16):
                d = dvec[j]
                x = (p0[p, r0 + j, :] + p1[p, r0 + j, :]) * d
                ac[p, r0 + j, :] = ac[p, r0 + j, :] + wl * x
                yc[p, r0 + j, :] = x * d
            return 0

        lax.fori_loop(0, _RCHUNK // 16, gbody, 0)
        sl = pl.ds(wid * _RPS + k * _RCHUNK, _RCHUNK)
        sdesc[p] = [pltpu.async_copy(yc.at[p], y_hbm.at[sl], osem),
                    pltpu.async_copy(ac.at[p], aggo_hbm.at[sl], osem)]
    for d_ in sdesc[(_NCH - 1) & 1]:
        d_.wait()


def _merge_call(l, pacc0, pacc1, dinv, agg, lw):
    return pl.kernel(
        functools.partial(_merge_body, l),
        out_type=(
            jax.ShapeDtypeStruct((_NPAD, _D), jnp.float32),
            jax.ShapeDtypeStruct((_NPAD, _D), jnp.float32),
        ),
        mesh=_mesh(),
        compiler_params=pltpu.CompilerParams(use_tc_tiling_on_sc=False, needs_layout_passes=False),
        scratch_types=[
            pltpu.VMEM((2, _RCHUNK, _D), jnp.float32),
            pltpu.VMEM((2, _RCHUNK, _D), jnp.float32),
            pltpu.VMEM((2, _RCHUNK, _D), jnp.float32),
            pltpu.VMEM((2, _RCHUNK, _D), jnp.float32),
            pltpu.VMEM((2, _RCHUNK), jnp.float32),
            pltpu.VMEM((LANES,), jnp.float32),
            pltpu.SemaphoreType.DMA,
            pltpu.SemaphoreType.DMA,
        ],
    )(pacc0, pacc1, dinv, agg, lw)


def _final_body(pacc0_hbm, pacc1_hbm, dinv_hbm, agg_hbm, lw_hbm, out_hbm,
                p0, p1, ac, yc, dv, lwv, lsem, osem):
    c, s, wid = _wid()
    pltpu.sync_copy(lw_hbm, lwv)
    wl = _softmax_weight(lwv[...], _L)

    def fire_loads(k, p):
        sl = pl.ds(wid * _RPS + k * _RCHUNK, _RCHUNK)
        return [pltpu.async_copy(pacc0_hbm.at[sl], p0.at[p], lsem),
                pltpu.async_copy(pacc1_hbm.at[sl], p1.at[p], lsem),
                pltpu.async_copy(dinv_hbm.at[sl], dv.at[p], lsem),
                pltpu.async_copy(agg_hbm.at[sl], ac.at[p], lsem)]

    ldesc = [None, None]
    sdesc = [None, None]
    ldesc[0] = fire_loads(0, 0)
    for k in range(_NCH):
        p = k & 1
        for d_ in ldesc[p]:
            d_.wait()
        if k >= 1:
            for d_ in sdesc[1 - p]:
                d_.wait()
        if k + 1 < _NCH:
            ldesc[1 - p] = fire_loads(k + 1, 1 - p)

        def gbody(g, _):
            r0 = g * 16
            dvec = dv[p, pl.ds(r0, 16)]
            for j in range(16):
                d = dvec[j]
                x = (p0[p, r0 + j, :] + p1[p, r0 + j, :]) * d
                a = ac[p, r0 + j, :] + wl * x
                ss = _lane_sum_scalar(a * a)
                ssv = jnp.broadcast_to(jnp.maximum(ss, 1e-24), (16,))
                yc[p, r0 + j, :] = a * _vrsqrt(ssv)
            return 0

        lax.fori_loop(0, _RCHUNK // 16, gbody, 0)
        sl = pl.ds(wid * _RPS + k * _RCHUNK, _RCHUNK)
        sdesc[p] = [pltpu.async_copy(yc.at[p], out_hbm.at[sl], osem)]
    for d_ in sdesc[(_NCH - 1) & 1]:
        d_.wait()


def _final_call(pacc0, pacc1, dinv, agg, lw):
    return pl.kernel(
        _final_body,
        out_type=jax.ShapeDtypeStruct((_NPAD, _D), jnp.float32),
        mesh=_mesh(),
        compiler_params=pltpu.CompilerParams(use_tc_tiling_on_sc=False, needs_layout_passes=False),
        scratch_types=[
            pltpu.VMEM((2, _RCHUNK, _D), jnp.float32),
            pltpu.VMEM((2, _RCHUNK, _D), jnp.float32),
            pltpu.VMEM((2, _RCHUNK, _D), jnp.float32),
            pltpu.VMEM((2, _RCHUNK, _D), jnp.float32),
            pltpu.VMEM((2, _RCHUNK), jnp.float32),
            pltpu.VMEM((LANES,), jnp.float32),
            pltpu.SemaphoreType.DMA,
            pltpu.SemaphoreType.DMA,
        ],
    )(pacc0, pacc1, dinv, agg, lw)


def kernel(edge_index, embedding_weight, layer_weights):
    src2 = edge_index[0].reshape(_NBLK, BLK)
    dst2 = edge_index[1].reshape(_NBLK, BLK)
    x0 = jnp.zeros((_NPAD, _D), jnp.float32).at[:_N].set(embedding_weight)
    lw = jnp.pad(layer_weights.astype(jnp.float32),
                 (0, LANES - layer_weights.shape[0]))
    pdeg0, pdeg1 = _deg_call(dst2)
    dinv, y, agg = _prep_call(pdeg0, pdeg1, x0, lw)
    out = None
    for l in range(1, _L + 1):
        pacc0, pacc1 = _edge_call(src2, dst2, y)
        if l < _L:
            y, agg = _merge_call(l, pacc0, pacc1, dinv, agg, lw)
        else:
            out = _final_call(pacc0, pacc1, dinv, agg, lw)
    return out[:_N]
